# baseline (speedup 1.0000x reference)
import functools

import jax
import jax.numpy as jnp
from jax import lax
from jax.experimental import pallas as pl
from jax.experimental.pallas import tpu as pltpu
from jax.experimental.pallas import tpu_sc as plsc

_BN_EPS = 1e-3
_G = 2


def _make_sc_pool(nsp, vocab, emb, batch, nc, ns):
    nw = nc * ns
    bpw = batch // nw
    rpp = 128 // emb
    nspp = (nsp + 7) // 8 * 8
    ngrp = (nsp + _G - 1) // _G
    groups = [list(range(g * _G, min((g + 1) * _G, nsp))) for g in range(ngrp)]

    mesh = plsc.VectorSubcoreMesh(core_axis_name="c", subcore_axis_name="s")

    @functools.partial(
        pl.kernel,
        mesh=mesh,
        compiler_params=pltpu.CompilerParams(needs_layout_passes=False),
        out_type=jax.ShapeDtypeStruct((batch, emb), jnp.float32),
        scratch_types=[
            pltpu.VMEM((nspp, bpw), jnp.int32),
            pltpu.VMEM((nspp, bpw), jnp.int32),
            [pltpu.VMEM((bpw, 128), jnp.float32) for _ in range(2 * _G)],
            pltpu.VMEM((bpw, emb), jnp.float32),
            pltpu.VMEM((bpw, emb), jnp.float32),
            pltpu.VMEM((bpw, emb), jnp.float32),
            pltpu.SemaphoreType.DMA,
            pltpu.SemaphoreType.DMA,
        ],
    )
    def sc_pool(idx_hbm, table_hbm, out_hbm, lane_v, blk_v, bufs,
                acc_v, acc2_v, out_v, sem0, sem1):
        sems = [sem0, sem1]
        wid = lax.axis_index("s") * nc + lax.axis_index("c")
        base = wid * bpw
        pltpu.sync_copy(idx_hbm.at[wid], lane_v)
        for f in range(nsp):
            foff = jnp.int32(f * (vocab // rpp))
            for c in range(bpw // 16):
                sl = pl.ds(c * 16, 16)
                raw = lane_v[f, sl]
                blk_v[f, sl] = lax.shift_right_logical(raw, 3) + foff
                lane_v[f, sl] = lax.bitwise_and(raw, rpp - 1) * emb

        iota16 = lax.iota(jnp.int32, 16)

        def fire(gi):
            fs = groups[gi]
            par = gi % 2
            return [
                pltpu.async_copy(table_hbm.at[blk_v.at[f]],
                                 bufs[par * _G + j], sems[par])
                for j, f in enumerate(fs)
            ]

        pending = fire(0)
        for gi, fs in enumerate(groups):
            for cp in pending:
                cp.wait()
            if gi + 1 < ngrp:
                pending = fire(gi + 1)
            par = gi % 2
            first = gi == 0
            last = gi == ngrp - 1

            def body(b, carry, fs=fs, par=par, first=first, last=last):
                i0 = jnp.full((16,), b, jnp.int32)
                vs = []
                for j, f in enumerate(fs):
                    msp = plsc.load_gather(
                        lane_v, [jnp.full((16,), f, jnp.int32), i0])
                    vs.append(plsc.load_gather(
                        bufs[par * _G + j], [i0, msp + iota16]))
                if first:
                    acc = vs[0]
                    acc2 = vs[0] * vs[0]
                    vs = vs[1:]
                else:
                    acc = acc_v[b]
                    acc2 = acc2_v[b]
                for v in vs:
                    acc = acc + v
                    acc2 = acc2 + v * v
                if last:
                    out_v[b] = 0.5 * (acc * acc - acc2)
                else:
                    acc_v[b] = acc
                    acc2_v[b] = acc2
                return carry

            lax.fori_loop(0, bpw, body, 0)

        pltpu.sync_copy(out_v, out_hbm.at[pl.ds(base, bpw)])

    return sc_pool


def _tc_mlp(in_ref, bi_ref, g_ref, be_ref, w1, b1, w2, b2, w3, b3, w4, b4,
            wo, bo, out_ref, *, ndense):
    dense = in_ref[...][:, :ndense]
    x = jnp.concatenate([dense, bi_ref[...]], axis=1)
    mean = jnp.mean(x, axis=0, keepdims=True)
    xc = x - mean
    var = jnp.mean(xc * xc, axis=0, keepdims=True)
    x = xc * lax.rsqrt(var + _BN_EPS) * g_ref[...] + be_ref[...]
    hp = jax.lax.Precision.HIGHEST
    x = jnp.maximum(jnp.dot(x, w1[...], precision=hp) + b1[...], 0.0)
    x = jnp.maximum(jnp.dot(x, w2[...], precision=hp) + b2[...], 0.0)
    x = jnp.maximum(jnp.dot(x, w3[...], precision=hp) + b3[...], 0.0)
    x = jnp.dot(x, w4[...], precision=hp) + b4[...]
    logit = jnp.dot(x, wo[...], precision=hp) + bo[...]
    out_ref[...] = jax.nn.sigmoid(logit)


def kernel(inputs, tables, gamma, beta, W1, b1, W2, b2, W3, b3, W4, b4, Wo, bo):
    batch, nfeat = inputs.shape
    nsp, vocab, emb = tables.shape
    ndense = nfeat - nsp

    info = plsc.get_sparse_core_info()
    nc, ns = info.num_cores, info.num_subcores
    nw = nc * ns
    bpw = batch // nw

    idx = inputs[:, ndense:].astype(jnp.int32)
    idx = idx.reshape(nw, bpw, nsp).transpose(0, 2, 1)
    nspp = (nsp + 7) // 8 * 8
    idx = jnp.pad(idx, ((0, 0), (0, nspp - nsp), (0, 0)))
    packed = tables.reshape(nsp * vocab * emb // 128, 128)

    bi = _make_sc_pool(nsp, vocab, emb, batch, nc, ns)(idx, packed)

    out = pl.pallas_call(
        functools.partial(_tc_mlp, ndense=ndense),
        out_shape=jax.ShapeDtypeStruct((batch, 1), jnp.float32),
    )(inputs, bi, gamma.reshape(1, -1), beta.reshape(1, -1),
      W1, b1.reshape(1, -1), W2, b2.reshape(1, -1), W3, b3.reshape(1, -1),
      W4, b4.reshape(1, -1), Wo, bo.reshape(1, 1))
    return out


# speedup vs baseline: 1.0379x; 1.0379x over previous
import functools

import jax
import jax.numpy as jnp
from jax import lax
from jax.experimental import pallas as pl
from jax.experimental.pallas import tpu as pltpu
from jax.experimental.pallas import tpu_sc as plsc

_BN_EPS = 1e-3


def _make_sc_pool(nsp, vocab, emb, batch, nc, ns):
    nw = nc * ns
    bpw = batch // nw

    mesh = plsc.VectorSubcoreMesh(core_axis_name="c", subcore_axis_name="s")

    @functools.partial(
        pl.kernel,
        mesh=mesh,
        compiler_params=pltpu.CompilerParams(use_tc_tiling_on_sc=False),
        out_type=jax.ShapeDtypeStruct((batch, emb), jnp.float32),
        scratch_types=[
            pltpu.VMEM((nsp, bpw), jnp.int32),
            pltpu.VMEM((nsp, bpw, emb), jnp.float32),
            pltpu.VMEM((bpw, emb), jnp.float32),
            pltpu.SemaphoreType.DMA,
        ],
    )
    def sc_pool(idx_hbm, table_hbm, out_hbm, idx_v, rows_v, out_v, sem):
        wid = lax.axis_index("s") * nc + lax.axis_index("c")
        base = wid * bpw
        pltpu.sync_copy(idx_hbm.at[wid], idx_v)
        copies = [
            pltpu.async_copy(table_hbm.at[f].at[idx_v.at[f]], rows_v.at[f],
                             sem)
            for f in range(nsp)
        ]
        for cp in copies:
            cp.wait()

        def body(b, carry):
            acc = rows_v[0, b]
            acc2 = acc * acc
            for f in range(1, nsp):
                v = rows_v[f, b]
                acc = acc + v
                acc2 = acc2 + v * v
            out_v[b] = 0.5 * (acc * acc - acc2)
            return carry

        lax.fori_loop(0, bpw, body, 0)
        pltpu.sync_copy(out_v, out_hbm.at[pl.ds(base, bpw)])

    return sc_pool


def _tc_mlp(in_ref, bi_ref, g_ref, be_ref, w1, b1, w2, b2, w3, b3, w4, b4,
            wo, bo, out_ref, *, ndense):
    dense = in_ref[...][:, :ndense]
    x = jnp.concatenate([dense, bi_ref[...]], axis=1)
    mean = jnp.mean(x, axis=0, keepdims=True)
    xc = x - mean
    var = jnp.mean(xc * xc, axis=0, keepdims=True)
    x = xc * lax.rsqrt(var + _BN_EPS) * g_ref[...] + be_ref[...]
    hp = jax.lax.Precision.HIGHEST
    x = jnp.maximum(jnp.dot(x, w1[...], precision=hp) + b1[...], 0.0)
    x = jnp.maximum(jnp.dot(x, w2[...], precision=hp) + b2[...], 0.0)
    x = jnp.maximum(jnp.dot(x, w3[...], precision=hp) + b3[...], 0.0)
    x = jnp.dot(x, w4[...], precision=hp) + b4[...]
    logit = jnp.dot(x, wo[...], precision=hp) + bo[...]
    out_ref[...] = jax.nn.sigmoid(logit)


def kernel(inputs, tables, gamma, beta, W1, b1, W2, b2, W3, b3, W4, b4, Wo, bo):
    batch, nfeat = inputs.shape
    nsp, vocab, emb = tables.shape
    ndense = nfeat - nsp

    info = plsc.get_sparse_core_info()
    nc, ns = info.num_cores, info.num_subcores
    nw = nc * ns
    bpw = batch // nw

    idx = inputs[:, ndense:].astype(jnp.int32)
    idx = idx.reshape(nw, bpw, nsp).transpose(0, 2, 1)

    bi = _make_sc_pool(nsp, vocab, emb, batch, nc, ns)(idx, tables)

    out = pl.pallas_call(
        functools.partial(_tc_mlp, ndense=ndense),
        out_shape=jax.ShapeDtypeStruct((batch, 1), jnp.float32),
    )(inputs, bi, gamma.reshape(1, -1), beta.reshape(1, -1),
      W1, b1.reshape(1, -1), W2, b2.reshape(1, -1), W3, b3.reshape(1, -1),
      W4, b4.reshape(1, -1), Wo, bo.reshape(1, 1))
    return out


# speedup vs baseline: 3.2248x; 3.1069x over previous
import functools

import jax
import jax.numpy as jnp
from jax import lax
from jax.experimental import pallas as pl
from jax.experimental.pallas import tpu as pltpu
from jax.experimental.pallas import tpu_sc as plsc

_BN_EPS = 1e-3


def _make_sc_pool(nsp, vocab, emb, batch, nc, ns):
    nw = nc * ns
    bpw = batch // nw

    mesh = plsc.VectorSubcoreMesh(core_axis_name="c", subcore_axis_name="s")

    @functools.partial(
        pl.kernel,
        mesh=mesh,
        compiler_params=pltpu.CompilerParams(use_tc_tiling_on_sc=False),
        out_type=jax.ShapeDtypeStruct((emb, batch), jnp.float32),
        scratch_types=[
            pltpu.VMEM((nsp, bpw), jnp.int32),
            pltpu.VMEM((nsp, emb, bpw), jnp.float32),
            pltpu.VMEM((emb, bpw), jnp.float32),
            pltpu.SemaphoreType.DMA,
        ],
    )
    def sc_pool(idx_hbm, table_hbm, out_hbm, idx_v, vals_v, out_v, sem):
        wid = lax.axis_index("s") * nc + lax.axis_index("c")
        base = wid * bpw
        pltpu.sync_copy(idx_hbm.at[wid], idx_v)

        copies = []
        for f in range(nsp):
            for e in range(emb):
                copies.append(pltpu.async_copy(
                    table_hbm.at[f, e].at[idx_v.at[f]],
                    vals_v.at[f, e], sem))
        for cp in copies:
            cp.wait()

        def body(c, carry):
            sl = pl.ds(c * 16, 16)
            for e in range(emb):
                acc = vals_v[0, e, sl]
                acc2 = acc * acc
                for f in range(1, nsp):
                    v = vals_v[f, e, sl]
                    acc = acc + v
                    acc2 = acc2 + v * v
                out_v[e, sl] = 0.5 * (acc * acc - acc2)
            return carry

        lax.fori_loop(0, bpw // 16, body, 0)
        pltpu.sync_copy(out_v, out_hbm.at[:, pl.ds(base, bpw)])

    return sc_pool


def _tc_mlp(in_ref, bi_ref, g_ref, be_ref, w1, b1, w2, b2, w3, b3, w4, b4,
            wo, bo, out_ref, *, ndense):
    dense = in_ref[...][:, :ndense]
    x = jnp.concatenate([dense, bi_ref[...].T], axis=1)
    mean = jnp.mean(x, axis=0, keepdims=True)
    xc = x - mean
    var = jnp.mean(xc * xc, axis=0, keepdims=True)
    x = xc * lax.rsqrt(var + _BN_EPS) * g_ref[...] + be_ref[...]
    hp = jax.lax.Precision.HIGHEST
    x = jnp.maximum(jnp.dot(x, w1[...], precision=hp) + b1[...], 0.0)
    x = jnp.maximum(jnp.dot(x, w2[...], precision=hp) + b2[...], 0.0)
    x = jnp.maximum(jnp.dot(x, w3[...], precision=hp) + b3[...], 0.0)
    x = jnp.dot(x, w4[...], precision=hp) + b4[...]
    logit = jnp.dot(x, wo[...], precision=hp) + bo[...]
    out_ref[...] = jax.nn.sigmoid(logit)


def kernel(inputs, tables, gamma, beta, W1, b1, W2, b2, W3, b3, W4, b4, Wo, bo):
    batch, nfeat = inputs.shape
    nsp, vocab, emb = tables.shape
    ndense = nfeat - nsp

    info = plsc.get_sparse_core_info()
    nc, ns = info.num_cores, info.num_subcores
    nw = nc * ns
    bpw = batch // nw

    idx = inputs[:, ndense:].astype(jnp.int32)
    idx = idx.reshape(nw, bpw, nsp).transpose(0, 2, 1)
    tables_t = tables.transpose(0, 2, 1)

    bi_t = _make_sc_pool(nsp, vocab, emb, batch, nc, ns)(idx, tables_t)

    out = pl.pallas_call(
        functools.partial(_tc_mlp, ndense=ndense),
        out_shape=jax.ShapeDtypeStruct((batch, 1), jnp.float32),
    )(inputs, bi_t, gamma.reshape(1, -1), beta.reshape(1, -1),
      W1, b1.reshape(1, -1), W2, b2.reshape(1, -1), W3, b3.reshape(1, -1),
      W4, b4.reshape(1, -1), Wo, bo.reshape(1, 1))
    return out
